# baseline (speedup 1.0000x reference)
import functools

import jax
import jax.numpy as jnp
from jax import lax
from jax.experimental import pallas as pl
from jax.experimental.pallas import tpu as pltpu

_VMEM_LIMIT = 56 * 1024 * 1024
_PREC = lax.Precision.HIGHEST


def _dot(a, b):
    return jnp.dot(a, b, preferred_element_type=jnp.float32, precision=_PREC)


def _up_kernel(x_ref, xh_ref, wall_ref, scale_ref, shift_ref, o_ref,
               *, cout, nh):
    i = pl.program_id(1)
    x = x_ref[0]
    th, w, cin = x.shape
    halo = jnp.where(i < nh - 1, xh_ref[0], jnp.zeros_like(xh_ref[0]))
    xd = jnp.concatenate([x[1:], halo], axis=0)
    zc = jnp.zeros((th, 1, cin), jnp.float32)
    xr = jnp.concatenate([x[:, 1:], zc], axis=1)
    xdr = jnp.concatenate([xd[:, 1:], zc], axis=1)
    m = th * w
    wall = wall_ref[...]
    acc = _dot(x.reshape(m, cin), wall[:cin])
    acc += _dot(xr.reshape(m, cin), wall[cin:2 * cin])
    acc += _dot(xd.reshape(m, cin), wall[2 * cin:3 * cin])
    acc += _dot(xdr.reshape(m, cin), wall[3 * cin:])
    y = jnp.maximum(acc * scale_ref[...] + shift_ref[...], 0.0)
    y = y[:, :4 * cout].reshape(th, w, 2, 2, cout)
    y = y.transpose(0, 2, 1, 3, 4).reshape(2 * th, 2 * w, cout)
    o_ref[0] = y


def _upsample(x, wall, scale, shift, *, cout, th):
    n, h, w, cin = x.shape
    th = min(th, h)
    nh = h // th
    return pl.pallas_call(
        functools.partial(_up_kernel, cout=cout, nh=nh),
        out_shape=jax.ShapeDtypeStruct((n, 2 * h, 2 * w, cout), jnp.float32),
        grid=(n, nh),
        in_specs=[
            pl.BlockSpec((1, th, w, cin), lambda b, i: (b, i, 0, 0)),
            pl.BlockSpec((1, 1, w, cin),
                         lambda b, i: (b, jnp.minimum((i + 1) * th, h - 1), 0, 0)),
            pl.BlockSpec(wall.shape, lambda b, i: (0, 0)),
            pl.BlockSpec(scale.shape, lambda b, i: (0, 0)),
            pl.BlockSpec(shift.shape, lambda b, i: (0, 0)),
        ],
        out_specs=pl.BlockSpec((1, 2 * th, 2 * w, cout), lambda b, i: (b, i, 0, 0)),
        compiler_params=pltpu.CompilerParams(
            dimension_semantics=("parallel", "parallel"),
            vmem_limit_bytes=_VMEM_LIMIT),
    )(x, x, wall, scale, shift)


def _hconv(v, wp, c, w):
    rows = v.shape[0] - 2
    op = jnp.concatenate([v[0:rows], v[1:1 + rows], v[2:2 + rows]], axis=-1)
    return _dot(op.reshape(rows * w, 3 * c), wp).reshape(rows, w, c)


def _wconv(v, wp, c, w):
    r = v.shape[0]
    z = jnp.zeros((r, 1, c), v.dtype)
    vp = jnp.concatenate([z, v, z], axis=1)
    op = jnp.concatenate([vp[:, 0:w], vp[:, 1:1 + w], vp[:, 2:2 + w]], axis=-1)
    return _dot(op.reshape(r * w, 3 * c), wp).reshape(r, w, c)


def _rowmask(t, g_first, h):
    g = g_first + lax.broadcasted_iota(jnp.int32, (t.shape[0], 1, 1), 0)
    return jnp.where((g >= 0) & (g < h), t, 0.0)


def _nb_half(v, g_first, h, c, w, w1, w2, w3, w4, b1, s1, h1, b3, s2, h2):
    t = _hconv(v, w1[...], c, w)
    t = jnp.maximum(t + b1[0], 0.0)
    t = _wconv(t, w2[...], c, w)
    t = jnp.maximum(t * s1[0] + h1[0], 0.0)
    t = _rowmask(t, g_first + 1, h)
    t = _hconv(t, w3[...], c, w)
    t = jnp.maximum(t + b3[0], 0.0)
    t = _wconv(t, w4[...], c, w)
    t = t * s2[0] + h2[0]
    rows = t.shape[0]
    return jnp.maximum(t + v[2:2 + rows], 0.0)


def _nb2_kernel(xt_ref, x_ref, xb_ref,
                w1a, w2a, w3a, w4a, b1a, s1a, h1a, b3a, s2a, h2a,
                w1b, w2b, w3b, w4b, b1b, s1b, h1b, b3b, s2b, h2b,
                *rest, th, h, fuse_out, nclass):
    if fuse_out:
        ow_ref, os_ref, oh_ref, o_ref = rest
    else:
        (o_ref,) = rest
    i = pl.program_id(1)
    x = x_ref[0]
    c, w = x.shape[2], x.shape[1]
    xf = jnp.concatenate([xt_ref[0, 0], x, xb_ref[0, 0]], axis=0)
    g0 = i * th - 4
    xf = _rowmask(xf, g0, h)
    ya = _nb_half(xf, g0, h, c, w,
                  w1a, w2a, w3a, w4a, b1a, s1a, h1a, b3a, s2a, h2a)
    ya = _rowmask(ya, g0 + 2, h)
    yb = _nb_half(ya, g0 + 2, h, c, w,
                  w1b, w2b, w3b, w4b, b1b, s1b, h1b, b3b, s2b, h2b)
    if not fuse_out:
        o_ref[0] = yb
        return
    ybt = jnp.transpose(yb.reshape(th * w, c))
    owt = jnp.transpose(ow_ref[...])[:4 * nclass]
    ost = jnp.transpose(os_ref[...])[:4 * nclass]
    oht = jnp.transpose(oh_ref[...])[:4 * nclass]
    yo = _dot(owt, ybt) * ost + oht
    o_ref[0] = yo.reshape(2, 2, nclass, th, w)


def _nb_pair(x, pa, pb, *, th, out_params=None, nclass=20):
    n, h, w, c = x.shape
    th = min(th, h)
    nh = h // th
    g = th // 4
    ng = h // 4
    xg = x.reshape(n, ng, 4, w, c)
    w_spec = pl.BlockSpec((3 * c, c), lambda b, i: (0, 0))
    v_spec = pl.BlockSpec((1, c), lambda b, i: (0, 0))
    in_specs = [
        pl.BlockSpec((1, 1, 4, w, c),
                     lambda b, i: (b, jnp.maximum(i * g - 1, 0), 0, 0, 0)),
        pl.BlockSpec((1, th, w, c), lambda b, i: (b, i, 0, 0)),
        pl.BlockSpec((1, 1, 4, w, c),
                     lambda b, i: (b, jnp.minimum((i + 1) * g, ng - 1), 0, 0, 0)),
    ] + [w_spec, w_spec, w_spec, w_spec,
         v_spec, v_spec, v_spec, v_spec, v_spec, v_spec] * 2
    args = [xg, x, xg,
            pa["w1"], pa["w2"], pa["w3"], pa["w4"],
            pa["b1"], pa["s1"], pa["h1"], pa["b3"], pa["s2"], pa["h2"],
            pb["w1"], pb["w2"], pb["w3"], pb["w4"],
            pb["b1"], pb["s1"], pb["h1"], pb["b3"], pb["s2"], pb["h2"]]
    if out_params is None:
        out_shape = jax.ShapeDtypeStruct((n, h, w, c), jnp.float32)
        out_spec = pl.BlockSpec((1, th, w, c), lambda b, i: (b, i, 0, 0))
        fuse_out = False
    else:
        ow, osc, osh = out_params
        in_specs += [pl.BlockSpec(ow.shape, lambda b, i: (0, 0)),
                     pl.BlockSpec(osc.shape, lambda b, i: (0, 0)),
                     pl.BlockSpec(osh.shape, lambda b, i: (0, 0))]
        args += [ow, osc, osh]
        out_shape = jax.ShapeDtypeStruct((n, 2, 2, nclass, h, w), jnp.float32)
        out_spec = pl.BlockSpec((1, 2, 2, nclass, th, w),
                                lambda b, i: (b, 0, 0, 0, i, 0))
        fuse_out = True
    return pl.pallas_call(
        functools.partial(_nb2_kernel, th=th, h=h,
                          fuse_out=fuse_out, nclass=nclass),
        out_shape=out_shape,
        grid=(n, nh),
        in_specs=in_specs,
        out_specs=out_spec,
        compiler_params=pltpu.CompilerParams(
            dimension_semantics=("parallel", "parallel"),
            vmem_limit_bytes=_VMEM_LIMIT),
    )(*args)


def _nbp(w1, w2, w3, w4, b1, s1, h1, b3, s2, h2):
    return dict(w1=w1, w2=w2, w3=w3, w4=w4,
                b1=b1, s1=s1, h1=h1, b3=b3, s2=s2, h2=h2)


def kernel(x, up1_wall, up1_scale, up1_shift,
           nb1_w1, nb1_w2, nb1_w3, nb1_w4, nb1_b1, nb1_s1, nb1_h1, nb1_b3, nb1_s2, nb1_h2,
           nb2_w1, nb2_w2, nb2_w3, nb2_w4, nb2_b1, nb2_s1, nb2_h1, nb2_b3, nb2_s2, nb2_h2,
           up2_wall, up2_scale, up2_shift,
           nb3_w1, nb3_w2, nb3_w3, nb3_w4, nb3_b1, nb3_s1, nb3_h1, nb3_b3, nb3_s2, nb3_h2,
           nb4_w1, nb4_w2, nb4_w3, nb4_w4, nb4_b1, nb4_s1, nb4_h1, nb4_b3, nb4_s2, nb4_h2,
           out_wall, out_scale, out_shift):
    y = jnp.transpose(x, (0, 2, 3, 1)).astype(jnp.float32)
    y = _upsample(y, up1_wall, up1_scale, up1_shift, cout=64, th=16)
    y = _nb_pair(y,
                 _nbp(nb1_w1, nb1_w2, nb1_w3, nb1_w4, nb1_b1, nb1_s1, nb1_h1,
                      nb1_b3, nb1_s2, nb1_h2),
                 _nbp(nb2_w1, nb2_w2, nb2_w3, nb2_w4, nb2_b1, nb2_s1, nb2_h1,
                      nb2_b3, nb2_s2, nb2_h2),
                 th=16)
    y = _upsample(y, up2_wall, up2_scale, up2_shift, cout=16, th=16)
    y = _nb_pair(y,
                 _nbp(nb3_w1, nb3_w2, nb3_w3, nb3_w4, nb3_b1, nb3_s1, nb3_h1,
                      nb3_b3, nb3_s2, nb3_h2),
                 _nbp(nb4_w1, nb4_w2, nb4_w3, nb4_w4, nb4_b1, nb4_s1, nb4_h1,
                      nb4_b3, nb4_s2, nb4_h2),
                 th=16, out_params=(out_wall, out_scale, out_shift), nclass=20)
    n, _, _, nc, h, w = y.shape
    return jnp.transpose(y, (0, 3, 4, 1, 5, 2)).reshape(n, nc, 2 * h, 2 * w)


# speedup vs baseline: 3.4462x; 3.4462x over previous
import functools

import jax
import jax.numpy as jnp
from jax import lax
from jax.experimental import pallas as pl
from jax.experimental.pallas import tpu as pltpu

_VMEM_LIMIT = 56 * 1024 * 1024
_PREC = lax.Precision.DEFAULT


def _dot(a, b):
    return jnp.dot(a, b, preferred_element_type=jnp.float32, precision=_PREC)


def _up_kernel(x_ref, xh_ref, wall_ref, scale_ref, shift_ref, o_ref,
               *, cout, nh):
    i = pl.program_id(1)
    x = x_ref[0]
    th, w, cin = x.shape
    halo = jnp.where(i < nh - 1, xh_ref[0], jnp.zeros_like(xh_ref[0]))
    xd = jnp.concatenate([x[1:], halo], axis=0)
    zc = jnp.zeros((th, 1, cin), jnp.float32)
    xr = jnp.concatenate([x[:, 1:], zc], axis=1)
    xdr = jnp.concatenate([xd[:, 1:], zc], axis=1)
    m = th * w
    wall = wall_ref[...]
    acc = _dot(x.reshape(m, cin), wall[:cin])
    acc += _dot(xr.reshape(m, cin), wall[cin:2 * cin])
    acc += _dot(xd.reshape(m, cin), wall[2 * cin:3 * cin])
    acc += _dot(xdr.reshape(m, cin), wall[3 * cin:])
    y = jnp.maximum(acc * scale_ref[...] + shift_ref[...], 0.0)
    y = y[:, :4 * cout].reshape(th, w, 2, 2, cout)
    y = y.transpose(0, 2, 1, 3, 4).reshape(2 * th, 2 * w, cout)
    o_ref[0] = y


def _upsample(x, wall, scale, shift, *, cout, th):
    n, h, w, cin = x.shape
    th = min(th, h)
    nh = h // th
    return pl.pallas_call(
        functools.partial(_up_kernel, cout=cout, nh=nh),
        out_shape=jax.ShapeDtypeStruct((n, 2 * h, 2 * w, cout), jnp.float32),
        grid=(n, nh),
        in_specs=[
            pl.BlockSpec((1, th, w, cin), lambda b, i: (b, i, 0, 0)),
            pl.BlockSpec((1, 1, w, cin),
                         lambda b, i: (b, jnp.minimum((i + 1) * th, h - 1), 0, 0)),
            pl.BlockSpec(wall.shape, lambda b, i: (0, 0)),
            pl.BlockSpec(scale.shape, lambda b, i: (0, 0)),
            pl.BlockSpec(shift.shape, lambda b, i: (0, 0)),
        ],
        out_specs=pl.BlockSpec((1, 2 * th, 2 * w, cout), lambda b, i: (b, i, 0, 0)),
        compiler_params=pltpu.CompilerParams(
            dimension_semantics=("parallel", "parallel"),
            vmem_limit_bytes=_VMEM_LIMIT),
    )(x, x, wall, scale, shift)


def _hconv(v, wp, c, w):
    rows = v.shape[0] - 2
    op = jnp.concatenate([v[0:rows], v[1:1 + rows], v[2:2 + rows]], axis=-1)
    return _dot(op.reshape(rows * w, 3 * c), wp).reshape(rows, w, c)


def _wconv(v, wp, c, w):
    r = v.shape[0]
    z = jnp.zeros((r, 1, c), v.dtype)
    vp = jnp.concatenate([z, v, z], axis=1)
    op = jnp.concatenate([vp[:, 0:w], vp[:, 1:1 + w], vp[:, 2:2 + w]], axis=-1)
    return _dot(op.reshape(r * w, 3 * c), wp).reshape(r, w, c)


def _rowmask(t, g_first, h):
    g = g_first + lax.broadcasted_iota(jnp.int32, (t.shape[0], 1, 1), 0)
    return jnp.where((g >= 0) & (g < h), t, 0.0)


def _nb_half(v, g_first, h, c, w, w1, w2, w3, w4, b1, s1, h1, b3, s2, h2):
    t = _hconv(v, w1[...], c, w)
    t = jnp.maximum(t + b1[0], 0.0)
    t = _wconv(t, w2[...], c, w)
    t = jnp.maximum(t * s1[0] + h1[0], 0.0)
    t = _rowmask(t, g_first + 1, h)
    t = _hconv(t, w3[...], c, w)
    t = jnp.maximum(t + b3[0], 0.0)
    t = _wconv(t, w4[...], c, w)
    t = t * s2[0] + h2[0]
    rows = t.shape[0]
    return jnp.maximum(t + v[2:2 + rows], 0.0)


def _nb2_kernel(xt_ref, x_ref, xb_ref,
                w1a, w2a, w3a, w4a, b1a, s1a, h1a, b3a, s2a, h2a,
                w1b, w2b, w3b, w4b, b1b, s1b, h1b, b3b, s2b, h2b,
                *rest, th, h, fuse_out, nclass):
    if fuse_out:
        ow_ref, os_ref, oh_ref, o_ref = rest
    else:
        (o_ref,) = rest
    i = pl.program_id(1)
    x = x_ref[0]
    c, w = x.shape[2], x.shape[1]
    xf = jnp.concatenate([xt_ref[0], x, xb_ref[0]], axis=0)
    g0 = i * th - 4
    xf = _rowmask(xf, g0, h)
    ya = _nb_half(xf, g0, h, c, w,
                  w1a, w2a, w3a, w4a, b1a, s1a, h1a, b3a, s2a, h2a)
    ya = _rowmask(ya, g0 + 2, h)
    yb = _nb_half(ya, g0 + 2, h, c, w,
                  w1b, w2b, w3b, w4b, b1b, s1b, h1b, b3b, s2b, h2b)
    if not fuse_out:
        o_ref[0] = yb
        return
    ybt = jnp.transpose(yb.reshape(th * w, c))
    owt = jnp.transpose(ow_ref[...])[:4 * nclass]
    ost = jnp.transpose(os_ref[...])[:4 * nclass]
    oht = jnp.transpose(oh_ref[...])[:4 * nclass]
    yo = _dot(owt, ybt) * ost + oht
    o_ref[0] = yo.reshape(2, 2, nclass, th, w)


def _nb_pair(x, pa, pb, *, th, out_params=None, nclass=20):
    n, h, w, c = x.shape
    th = min(th, h)
    nh = h // th
    g = th // 4
    ng = h // 4
    w_spec = pl.BlockSpec((3 * c, c), lambda b, i: (0, 0))
    v_spec = pl.BlockSpec((1, c), lambda b, i: (0, 0))
    in_specs = [
        pl.BlockSpec((1, 4, w, c),
                     lambda b, i: (b, jnp.maximum(i * g - 1, 0), 0, 0)),
        pl.BlockSpec((1, th, w, c), lambda b, i: (b, i, 0, 0)),
        pl.BlockSpec((1, 4, w, c),
                     lambda b, i: (b, jnp.minimum((i + 1) * g, ng - 1), 0, 0)),
    ] + [w_spec, w_spec, w_spec, w_spec,
         v_spec, v_spec, v_spec, v_spec, v_spec, v_spec] * 2
    args = [x, x, x,
            pa["w1"], pa["w2"], pa["w3"], pa["w4"],
            pa["b1"], pa["s1"], pa["h1"], pa["b3"], pa["s2"], pa["h2"],
            pb["w1"], pb["w2"], pb["w3"], pb["w4"],
            pb["b1"], pb["s1"], pb["h1"], pb["b3"], pb["s2"], pb["h2"]]
    if out_params is None:
        out_shape = jax.ShapeDtypeStruct((n, h, w, c), jnp.float32)
        out_spec = pl.BlockSpec((1, th, w, c), lambda b, i: (b, i, 0, 0))
        fuse_out = False
    else:
        ow, osc, osh = out_params
        in_specs += [pl.BlockSpec(ow.shape, lambda b, i: (0, 0)),
                     pl.BlockSpec(osc.shape, lambda b, i: (0, 0)),
                     pl.BlockSpec(osh.shape, lambda b, i: (0, 0))]
        args += [ow, osc, osh]
        out_shape = jax.ShapeDtypeStruct((n, 2, 2, nclass, h, w), jnp.float32)
        out_spec = pl.BlockSpec((1, 2, 2, nclass, th, w),
                                lambda b, i: (b, 0, 0, 0, i, 0))
        fuse_out = True
    return pl.pallas_call(
        functools.partial(_nb2_kernel, th=th, h=h,
                          fuse_out=fuse_out, nclass=nclass),
        out_shape=out_shape,
        grid=(n, nh),
        in_specs=in_specs,
        out_specs=out_spec,
        compiler_params=pltpu.CompilerParams(
            dimension_semantics=("parallel", "parallel"),
            vmem_limit_bytes=_VMEM_LIMIT),
    )(*args)


def _nbp(w1, w2, w3, w4, b1, s1, h1, b3, s2, h2):
    return dict(w1=w1, w2=w2, w3=w3, w4=w4,
                b1=b1, s1=s1, h1=h1, b3=b3, s2=s2, h2=h2)


def kernel(x, up1_wall, up1_scale, up1_shift,
           nb1_w1, nb1_w2, nb1_w3, nb1_w4, nb1_b1, nb1_s1, nb1_h1, nb1_b3, nb1_s2, nb1_h2,
           nb2_w1, nb2_w2, nb2_w3, nb2_w4, nb2_b1, nb2_s1, nb2_h1, nb2_b3, nb2_s2, nb2_h2,
           up2_wall, up2_scale, up2_shift,
           nb3_w1, nb3_w2, nb3_w3, nb3_w4, nb3_b1, nb3_s1, nb3_h1, nb3_b3, nb3_s2, nb3_h2,
           nb4_w1, nb4_w2, nb4_w3, nb4_w4, nb4_b1, nb4_s1, nb4_h1, nb4_b3, nb4_s2, nb4_h2,
           out_wall, out_scale, out_shift):
    y = jnp.transpose(x, (0, 2, 3, 1)).astype(jnp.float32)
    y = _upsample(y, up1_wall, up1_scale, up1_shift, cout=64, th=16)
    y = _nb_pair(y,
                 _nbp(nb1_w1, nb1_w2, nb1_w3, nb1_w4, nb1_b1, nb1_s1, nb1_h1,
                      nb1_b3, nb1_s2, nb1_h2),
                 _nbp(nb2_w1, nb2_w2, nb2_w3, nb2_w4, nb2_b1, nb2_s1, nb2_h1,
                      nb2_b3, nb2_s2, nb2_h2),
                 th=16)
    y = _upsample(y, up2_wall, up2_scale, up2_shift, cout=16, th=16)
    y = _nb_pair(y,
                 _nbp(nb3_w1, nb3_w2, nb3_w3, nb3_w4, nb3_b1, nb3_s1, nb3_h1,
                      nb3_b3, nb3_s2, nb3_h2),
                 _nbp(nb4_w1, nb4_w2, nb4_w3, nb4_w4, nb4_b1, nb4_s1, nb4_h1,
                      nb4_b3, nb4_s2, nb4_h2),
                 th=16, out_params=(out_wall, out_scale, out_shift), nclass=20)
    n, _, _, nc, h, w = y.shape
    return jnp.transpose(y, (0, 3, 4, 1, 5, 2)).reshape(n, nc, 2 * h, 2 * w)
